# Initial kernel scaffold; baseline (speedup 1.0000x reference)
import functools

import jax
import jax.numpy as jnp
from jax import lax
from jax.experimental import pallas as pl
from jax.experimental.pallas import tpu as pltpu
from jax.experimental.pallas import tpu_sc as plsc

V = 1000000
D = 64
B = 4096
T = 204800
H1 = 512
H2 = 256
OUT = 100
EPS = 1e-5

NC = 2
NS = 16
L = 16
NW = NC * NS
TPW = T // NW
CH = 128
NCH = TPW // CH
RPT = B // NS
OFF_PAD = 8192
OUT_PAD = 128


def _make_sc_pool():
    mesh = plsc.VectorSubcoreMesh(core_axis_name="c", subcore_axis_name="s")

    @functools.partial(
        pl.kernel,
        mesh=mesh,
        out_type=jax.ShapeDtypeStruct((NC * B, D), jnp.float32),
        scratch_types=[
            pltpu.VMEM((OFF_PAD,), jnp.int32),
            pltpu.VMEM((NCH, CH), jnp.int32),
            pltpu.VMEM((NCH, CH), jnp.int32),
            pltpu.VMEM((CH, D), jnp.float32),
            pltpu.VMEM((RPT, D), jnp.float32),
            pltpu.VMEM_SHARED((B, D), jnp.float32),
            pltpu.SemaphoreType.DMA,
        ],
    )
    def sc_pool(tok_hbm, off_hbm, table_hbm, out_hbm,
                off_v, tok_v, seg_v, rows_v, tmp_v, acc, sem):
        cid = lax.axis_index("c")
        sid = lax.axis_index("s")
        wid = sid * NC + cid

        pltpu.sync_copy(off_hbm, off_v)
        tok_cp = pltpu.async_copy(tok_hbm.at[pl.ds(wid * NCH, NCH)], tok_v, sem)

        zero = jnp.zeros((L,), jnp.float32)

        def zrow(i, carry):
            for k in range(D // L):
                tmp_v[i, pl.ds(k * L, L)] = zero
            return carry

        lax.fori_loop(0, RPT, zrow, 0)
        pltpu.sync_copy(tmp_v, acc.at[pl.ds(sid * RPT, RPT)])

        lane = lax.iota(jnp.int32, L)
        base0 = wid * TPW

        def seg_row(j, carry):
            for k in range(CH // L):
                pos = base0 + j * CH + (k * L) + lane
                ans = jnp.zeros((L,), jnp.int32)
                step = B // 2
                while step >= 1:
                    cand = ans + step
                    vals = plsc.load_gather(off_v, [cand])
                    ans = jnp.where(vals <= pos, cand, ans)
                    step //= 2
                seg_v[j, pl.ds(k * L, L)] = ans
            return carry

        lax.fori_loop(0, NCH, seg_row, 0)

        tok_cp.wait()
        plsc.subcore_barrier()

        def chunk(j, carry):
            pltpu.async_copy(table_hbm.at[tok_v.at[j]], rows_v, sem).wait()
            pltpu.sync_copy(rows_v, acc.at[seg_v.at[j]], add=True)
            return carry

        lax.fori_loop(0, NCH, chunk, 0)

        plsc.subcore_barrier()

        pltpu.sync_copy(acc.at[pl.ds(sid * RPT, RPT)], tmp_v)
        pltpu.sync_copy(tmp_v, out_hbm.at[pl.ds(cid * B + sid * RPT, RPT)])

    return sc_pool


_sc_pool = _make_sc_pool()


def _layer_norm(x, g, b):
    mu = jnp.mean(x, axis=-1, keepdims=True)
    d = x - mu
    var = jnp.mean(d * d, axis=-1, keepdims=True)
    return d * lax.rsqrt(var + EPS) * g + b


def _tower_body(p0_ref, p1_ref, lo_ref, hi_ref, W1_ref, b1_ref, g1_ref,
                be1_ref, W2_ref, b2_ref, g2_ref, be2_ref, Wo_ref, bo_ref,
                out_ref):
    cnt = (hi_ref[...] - lo_ref[...]).astype(jnp.float32)
    inv = 1.0 / jnp.maximum(cnt, 1.0)
    x = (p0_ref[...] + p1_ref[...]) * inv
    h = jnp.dot(x, W1_ref[...], preferred_element_type=jnp.float32)
    h = _layer_norm(h + b1_ref[...], g1_ref[...], be1_ref[...])
    h = jnp.maximum(h, 0.0)
    h = jnp.dot(h, W2_ref[...], preferred_element_type=jnp.float32)
    h = _layer_norm(h + b2_ref[...], g2_ref[...], be2_ref[...])
    h = jnp.maximum(h, 0.0)
    out_ref[...] = (
        jnp.dot(h, Wo_ref[...], preferred_element_type=jnp.float32)
        + bo_ref[...]
    )


_BLK = 512


def _tower(p0, p1, off_lo, off_hi, W1, b1, g1, be1, W2, b2, g2, be2, Wo, bo):
    full = lambda shape: pl.BlockSpec(shape, lambda i: (0, 0))
    return pl.pallas_call(
        _tower_body,
        grid=(B // _BLK,),
        in_specs=[
            pl.BlockSpec((_BLK, D), lambda i: (i, 0)),
            pl.BlockSpec((_BLK, D), lambda i: (i, 0)),
            pl.BlockSpec((_BLK, 1), lambda i: (i, 0)),
            pl.BlockSpec((_BLK, 1), lambda i: (i, 0)),
            full((D, H1)), full((1, H1)), full((1, H1)), full((1, H1)),
            full((H1, H2)), full((1, H2)), full((1, H2)), full((1, H2)),
            full((H2, OUT_PAD)), full((1, OUT_PAD)),
        ],
        out_specs=pl.BlockSpec((_BLK, OUT_PAD), lambda i: (i, 0)),
        out_shape=jax.ShapeDtypeStruct((B, OUT_PAD), jnp.float32),
    )(p0, p1, off_lo, off_hi, W1, b1, g1, be1, W2, b2, g2, be2, Wo, bo)


def kernel(flattened_tokens, offsets, table,
           W1, b1, g1, be1, W2, b2, g2, be2, Wo, bo):
    tok2d = flattened_tokens.reshape(T // CH, CH)
    off_pad = jnp.concatenate(
        [offsets,
         jnp.full((OFF_PAD - B,), jnp.iinfo(jnp.int32).max, jnp.int32)])
    partials = _sc_pool(tok2d, off_pad, table)

    off_lo = offsets.reshape(B, 1)
    off_hi = jnp.concatenate(
        [offsets[1:], jnp.array([T], jnp.int32)]).reshape(B, 1)
    Wo_p = jnp.pad(Wo, ((0, 0), (0, OUT_PAD - OUT)))
    bo_p = jnp.pad(bo, (0, OUT_PAD - OUT)).reshape(1, OUT_PAD)
    out = _tower(partials[:B], partials[B:], off_lo, off_hi,
                 W1, b1.reshape(1, H1), g1.reshape(1, H1), be1.reshape(1, H1),
                 W2, b2.reshape(1, H2), g2.reshape(1, H2), be2.reshape(1, H2),
                 Wo_p, bo_p)
    return out[:, :OUT]


# speedup vs baseline: 28.6648x; 28.6648x over previous
import functools

import jax
import jax.numpy as jnp
from jax import lax
from jax.experimental import pallas as pl
from jax.experimental.pallas import tpu as pltpu
from jax.experimental.pallas import tpu_sc as plsc

V = 1000000
D = 64
B = 4096
T = 204800
H1 = 512
H2 = 256
OUT = 100
EPS = 1e-5

NC = 2
NS = 16
L = 16
NW = NC * NS
TPW = T // NW
CH = 128
NCH = TPW // CH
RPT = B // NS
OFF_PAD = 8192
OUT_PAD = 128


def _make_sc_pool():
    mesh = plsc.VectorSubcoreMesh(core_axis_name="c", subcore_axis_name="s")

    @functools.partial(
        pl.kernel,
        mesh=mesh,
        out_type=jax.ShapeDtypeStruct((NC * B, D), jnp.float32),
        compiler_params=pltpu.CompilerParams(
            needs_layout_passes=False, use_tc_tiling_on_sc=False),
        scratch_types=[
            pltpu.VMEM((OFF_PAD,), jnp.int32),
            pltpu.VMEM((TPW,), jnp.int32),
            pltpu.VMEM((NCH, CH), jnp.int32),
            pltpu.VMEM((CH, D), jnp.float32),
            pltpu.VMEM((RPT, D), jnp.float32),
            pltpu.VMEM_SHARED((B, D), jnp.float32),
            pltpu.SemaphoreType.DMA,
        ],
    )
    def sc_pool(tok_hbm, off_hbm, table_hbm, out_hbm,
                off_v, tok_v, seg_v, rows_v, tmp_v, acc, sem):
        cid = lax.axis_index("c")
        sid = lax.axis_index("s")
        wid = sid * NC + cid

        pltpu.sync_copy(off_hbm, off_v)
        tok_cp = pltpu.async_copy(tok_hbm.at[pl.ds(wid * TPW, TPW)], tok_v, sem)

        zero = jnp.zeros((L,), jnp.float32)

        def zrow(i, carry):
            for k in range(D // L):
                tmp_v[i, pl.ds(k * L, L)] = zero
            return carry

        lax.fori_loop(0, RPT, zrow, 0)
        pltpu.sync_copy(tmp_v, acc.at[pl.ds(sid * RPT, RPT)])

        lane = lax.iota(jnp.int32, L)
        base0 = wid * TPW

        def seg_row(j, carry):
            for k in range(CH // L):
                pos = base0 + j * CH + (k * L) + lane
                ans = jnp.zeros((L,), jnp.int32)
                step = B // 2
                while step >= 1:
                    cand = ans + step
                    vals = plsc.load_gather(off_v, [cand])
                    ans = jnp.where(vals <= pos, cand, ans)
                    step //= 2
                seg_v[j, pl.ds(k * L, L)] = ans
            return carry

        lax.fori_loop(0, NCH, seg_row, 0)

        tok_cp.wait()
        plsc.subcore_barrier()

        def chunk(j, carry):
            tbase = pl.multiple_of(j * CH, CH)
            pltpu.async_copy(table_hbm.at[tok_v.at[pl.ds(tbase, CH)]],
                             rows_v, sem).wait()
            pltpu.sync_copy(rows_v, acc.at[seg_v.at[j]], add=True)
            return carry

        lax.fori_loop(0, NCH, chunk, 0)

        plsc.subcore_barrier()

        pltpu.sync_copy(acc.at[pl.ds(sid * RPT, RPT)], tmp_v)
        pltpu.sync_copy(tmp_v, out_hbm.at[pl.ds(cid * B + sid * RPT, RPT)])

    return sc_pool


_sc_pool = _make_sc_pool()


def _layer_norm(x, g, b):
    mu = jnp.mean(x, axis=-1, keepdims=True)
    d = x - mu
    var = jnp.mean(d * d, axis=-1, keepdims=True)
    return d * lax.rsqrt(var + EPS) * g + b


def _tower_body(p0_ref, p1_ref, lo_ref, hi_ref, W1_ref, b1_ref, g1_ref,
                be1_ref, W2_ref, b2_ref, g2_ref, be2_ref, Wo_ref, bo_ref,
                out_ref):
    cnt = (hi_ref[...] - lo_ref[...]).astype(jnp.float32)
    inv = 1.0 / jnp.maximum(cnt, 1.0)
    x = (p0_ref[...] + p1_ref[...]) * inv
    h = jnp.dot(x, W1_ref[...], preferred_element_type=jnp.float32)
    h = _layer_norm(h + b1_ref[...], g1_ref[...], be1_ref[...])
    h = jnp.maximum(h, 0.0)
    h = jnp.dot(h, W2_ref[...], preferred_element_type=jnp.float32)
    h = _layer_norm(h + b2_ref[...], g2_ref[...], be2_ref[...])
    h = jnp.maximum(h, 0.0)
    out_ref[...] = (
        jnp.dot(h, Wo_ref[...], preferred_element_type=jnp.float32)
        + bo_ref[...]
    )


_BLK = 512


def _tower(p0, p1, off_lo, off_hi, W1, b1, g1, be1, W2, b2, g2, be2, Wo, bo):
    full = lambda shape: pl.BlockSpec(shape, lambda i: (0, 0))
    return pl.pallas_call(
        _tower_body,
        grid=(B // _BLK,),
        in_specs=[
            pl.BlockSpec((_BLK, D), lambda i: (i, 0)),
            pl.BlockSpec((_BLK, D), lambda i: (i, 0)),
            pl.BlockSpec((_BLK, 1), lambda i: (i, 0)),
            pl.BlockSpec((_BLK, 1), lambda i: (i, 0)),
            full((D, H1)), full((1, H1)), full((1, H1)), full((1, H1)),
            full((H1, H2)), full((1, H2)), full((1, H2)), full((1, H2)),
            full((H2, OUT_PAD)), full((1, OUT_PAD)),
        ],
        out_specs=pl.BlockSpec((_BLK, OUT_PAD), lambda i: (i, 0)),
        out_shape=jax.ShapeDtypeStruct((B, OUT_PAD), jnp.float32),
    )(p0, p1, off_lo, off_hi, W1, b1, g1, be1, W2, b2, g2, be2, Wo, bo)


def kernel(flattened_tokens, offsets, table,
           W1, b1, g1, be1, W2, b2, g2, be2, Wo, bo):
    off_pad = jnp.concatenate(
        [offsets,
         jnp.full((OFF_PAD - B,), jnp.iinfo(jnp.int32).max, jnp.int32)])
    partials = _sc_pool(flattened_tokens, off_pad, table)

    off_lo = offsets.reshape(B, 1)
    off_hi = jnp.concatenate(
        [offsets[1:], jnp.array([T], jnp.int32)]).reshape(B, 1)
    Wo_p = jnp.pad(Wo, ((0, 0), (0, OUT_PAD - OUT)))
    bo_p = jnp.pad(bo, (0, OUT_PAD - OUT)).reshape(1, OUT_PAD)
    out = _tower(partials[:B], partials[B:], off_lo, off_hi,
                 W1, b1.reshape(1, H1), g1.reshape(1, H1), be1.reshape(1, H1),
                 W2, b2.reshape(1, H2), g2.reshape(1, H2), be2.reshape(1, H2),
                 Wo_p, bo_p)
    return out[:, :OUT]


# speedup vs baseline: 30.6117x; 1.0679x over previous
import functools

import jax
import jax.numpy as jnp
from jax import lax
from jax.experimental import pallas as pl
from jax.experimental.pallas import tpu as pltpu
from jax.experimental.pallas import tpu_sc as plsc

V = 1000000
D = 64
B = 4096
T = 204800
H1 = 512
H2 = 256
OUT = 100
EPS = 1e-5

NC = 2
NS = 16
L = 16
NW = NC * NS
TPW = T // NW
CH = 128
NCH = TPW // CH
RPT = B // NS
OFF_PAD = 8192
OUT_PAD = 128


def _make_sc_pool():
    mesh = plsc.VectorSubcoreMesh(core_axis_name="c", subcore_axis_name="s")

    @functools.partial(
        pl.kernel,
        mesh=mesh,
        out_type=jax.ShapeDtypeStruct((NC * B, D), jnp.float32),
        compiler_params=pltpu.CompilerParams(
            needs_layout_passes=False, use_tc_tiling_on_sc=False),
        scratch_types=[
            pltpu.VMEM((OFF_PAD,), jnp.int32),
            pltpu.VMEM((TPW,), jnp.int32),
            pltpu.VMEM((NCH, CH), jnp.int32),
            pltpu.VMEM((CH, D), jnp.float32),
            pltpu.VMEM((CH, D), jnp.float32),
            pltpu.VMEM((RPT, D), jnp.float32),
            pltpu.VMEM_SHARED((B, D), jnp.float32),
            pltpu.SemaphoreType.DMA,
            pltpu.SemaphoreType.DMA,
            pltpu.SemaphoreType.DMA,
        ],
    )
    def sc_pool(tok_hbm, off_hbm, table_hbm, out_hbm,
                off_v, tok_v, seg_v, rows0_v, rows1_v, tmp_v, acc,
                sem, sem0, sem1):
        cid = lax.axis_index("c")
        sid = lax.axis_index("s")
        wid = sid * NC + cid

        pltpu.sync_copy(off_hbm, off_v)
        tok_cp = pltpu.async_copy(tok_hbm.at[pl.ds(wid * TPW, TPW)], tok_v, sem)

        zero = jnp.zeros((L,), jnp.float32)

        def zrow(i, carry):
            for k in range(D // L):
                tmp_v[i, pl.ds(k * L, L)] = zero
            return carry

        lax.fori_loop(0, RPT, zrow, 0)
        pltpu.sync_copy(tmp_v, acc.at[pl.ds(sid * RPT, RPT)])

        lane = lax.iota(jnp.int32, L)
        base0 = wid * TPW

        def seg_row(j):
            for k in range(CH // L):
                pos = base0 + j * CH + (k * L) + lane
                ans = jnp.zeros((L,), jnp.int32)
                step = B // 2
                while step >= 1:
                    cand = ans + step
                    vals = plsc.load_gather(off_v, [cand])
                    ans = jnp.where(vals <= pos, cand, ans)
                    step //= 2
                seg_v[j, pl.ds(k * L, L)] = ans

        def fire(j, buf, s):
            tbase = pl.multiple_of(j * CH, CH)
            pltpu.async_copy(table_hbm.at[tok_v.at[pl.ds(tbase, CH)]], buf, s)

        def drain(buf, s):
            pltpu.make_async_copy(table_hbm.at[pl.ds(0, CH)], buf, s).wait()

        def scat(j, buf):
            pltpu.sync_copy(buf, acc.at[seg_v.at[j]], add=True)

        tok_cp.wait()
        fire(0, rows0_v, sem0)
        plsc.subcore_barrier()

        def body(i, carry):
            j = 2 * i
            fire(j + 1, rows1_v, sem1)
            seg_row(j)
            drain(rows0_v, sem0)
            scat(j, rows0_v)

            @pl.when(j + 2 < NCH)
            def _():
                fire(j + 2, rows0_v, sem0)

            seg_row(j + 1)
            drain(rows1_v, sem1)
            scat(j + 1, rows1_v)
            return carry

        lax.fori_loop(0, NCH // 2, body, 0)

        plsc.subcore_barrier()

        pltpu.sync_copy(acc.at[pl.ds(sid * RPT, RPT)], tmp_v)
        pltpu.sync_copy(tmp_v, out_hbm.at[pl.ds(cid * B + sid * RPT, RPT)])

    return sc_pool


_sc_pool = _make_sc_pool()


def _layer_norm(x, g, b):
    mu = jnp.mean(x, axis=-1, keepdims=True)
    d = x - mu
    var = jnp.mean(d * d, axis=-1, keepdims=True)
    return d * lax.rsqrt(var + EPS) * g + b


def _tower_body(p0_ref, p1_ref, lo_ref, hi_ref, W1_ref, b1_ref, g1_ref,
                be1_ref, W2_ref, b2_ref, g2_ref, be2_ref, Wo_ref, bo_ref,
                out_ref):
    cnt = (hi_ref[...] - lo_ref[...]).astype(jnp.float32)
    inv = 1.0 / jnp.maximum(cnt, 1.0)
    x = (p0_ref[...] + p1_ref[...]) * inv
    h = jnp.dot(x, W1_ref[...], preferred_element_type=jnp.float32)
    h = _layer_norm(h + b1_ref[...], g1_ref[...], be1_ref[...])
    h = jnp.maximum(h, 0.0)
    h = jnp.dot(h, W2_ref[...], preferred_element_type=jnp.float32)
    h = _layer_norm(h + b2_ref[...], g2_ref[...], be2_ref[...])
    h = jnp.maximum(h, 0.0)
    out_ref[...] = (
        jnp.dot(h, Wo_ref[...], preferred_element_type=jnp.float32)
        + bo_ref[...]
    )


_BLK = 512


def _tower(p0, p1, off_lo, off_hi, W1, b1, g1, be1, W2, b2, g2, be2, Wo, bo):
    full = lambda shape: pl.BlockSpec(shape, lambda i: (0, 0))
    return pl.pallas_call(
        _tower_body,
        grid=(B // _BLK,),
        in_specs=[
            pl.BlockSpec((_BLK, D), lambda i: (i, 0)),
            pl.BlockSpec((_BLK, D), lambda i: (i, 0)),
            pl.BlockSpec((_BLK, 1), lambda i: (i, 0)),
            pl.BlockSpec((_BLK, 1), lambda i: (i, 0)),
            full((D, H1)), full((1, H1)), full((1, H1)), full((1, H1)),
            full((H1, H2)), full((1, H2)), full((1, H2)), full((1, H2)),
            full((H2, OUT_PAD)), full((1, OUT_PAD)),
        ],
        out_specs=pl.BlockSpec((_BLK, OUT_PAD), lambda i: (i, 0)),
        out_shape=jax.ShapeDtypeStruct((B, OUT_PAD), jnp.float32),
    )(p0, p1, off_lo, off_hi, W1, b1, g1, be1, W2, b2, g2, be2, Wo, bo)


def kernel(flattened_tokens, offsets, table,
           W1, b1, g1, be1, W2, b2, g2, be2, Wo, bo):
    off_pad = jnp.concatenate(
        [offsets,
         jnp.full((OFF_PAD - B,), jnp.iinfo(jnp.int32).max, jnp.int32)])
    partials = _sc_pool(flattened_tokens, off_pad, table)

    off_lo = offsets.reshape(B, 1)
    off_hi = jnp.concatenate(
        [offsets[1:], jnp.array([T], jnp.int32)]).reshape(B, 1)
    Wo_p = jnp.pad(Wo, ((0, 0), (0, OUT_PAD - OUT)))
    bo_p = jnp.pad(bo, (0, OUT_PAD - OUT)).reshape(1, OUT_PAD)
    out = _tower(partials[:B], partials[B:], off_lo, off_hi,
                 W1, b1.reshape(1, H1), g1.reshape(1, H1), be1.reshape(1, H1),
                 W2, b2.reshape(1, H2), g2.reshape(1, H2), be2.reshape(1, H2),
                 Wo_p, bo_p)
    return out[:, :OUT]
